# baseline (speedup 1.0000x reference)
import jax
import jax.numpy as jnp
import numpy as np
from jax import lax
from jax.experimental import pallas as pl
from jax.experimental.pallas import tpu as pltpu
from jax.experimental.pallas import tpu_sc as plsc

E = 8
K = 4
D_ENC = 256
D_LAT = 256
HID = 256
OUT = 4
DEPTH = 7
SKIP = 5
CH = D_LAT // E
EIN = D_ENC + CH
XL = D_ENC + D_LAT

T = 4096
TB = 512
B = 256
LOGB = 8
NB = T // B + E
NBP = 32
P = NB * B
FW = 128

NS = 16
TPW = T // NS
SPW = P // NS
QR = SPW // 4
L = 16

_EPS = np.float32(np.finfo(np.float32).eps)


def _dot(a, b):
    return jnp.dot(a, b, preferred_element_type=jnp.float32)


def _gate_body(x_ref, lat_ref, gW1, gb1, gW2, gb2, gln_g, gln_b, gW3, gb3,
               eidx_ref, gval_ref):
    xb = x_ref[...]
    lb = lat_ref[...]
    g = jax.nn.relu(_dot(xb, gW1[:D_ENC]) + _dot(lb, gW1[D_ENC:]) + gb1[...])
    g = _dot(g, gW2[...]) + gb2[...]
    m = g.mean(-1, keepdims=True)
    v = ((g - m) ** 2).mean(-1, keepdims=True)
    g = (g - m) / jnp.sqrt(v + 1e-5) * gln_g[...] + gln_b[...]
    logits = _dot(g, gW3[...]) + gb3[...]
    mx = logits.max(-1, keepdims=True)
    s = jnp.exp(logits - mx)
    s = s / s.sum(-1, keepdims=True)
    lanes = jax.lax.broadcasted_iota(jnp.int32, s.shape, 1)
    cur = s
    sum4 = jnp.zeros((s.shape[0], 1), jnp.float32)
    eidx = gmax = None
    for r in range(K):
        m_r = cur.max(-1, keepdims=True)
        i_r = jnp.where(cur == m_r, lanes, E).min(-1, keepdims=True)
        sum4 = sum4 + m_r
        if r == 0:
            eidx, gmax = i_r, m_r
        cur = jnp.where(lanes == i_r, -jnp.inf, cur)
    gval = gmax / (sum4 + 1e-9)
    eidx_ref[...] = eidx.reshape(1, TB, 1)
    gval_ref[...] = gval.reshape(1, TB, 1)


def _gate(x, latent, gW1, gb1, gW2, gb2, gln_g, gln_b, gW3, gb3):
    tok = lambda i: (i, 0)
    out3 = lambda i: (i, 0, 0)
    def wspec(a):
        return pl.BlockSpec(a.shape, lambda i, _a=a: tuple([0] * _a.ndim))
    eidx, gval = pl.pallas_call(
        _gate_body,
        grid=(T // TB,),
        in_specs=[pl.BlockSpec((TB, D_ENC), tok),
                  pl.BlockSpec((TB, D_LAT), tok)]
                 + [wspec(a) for a in (gW1, gb1, gW2, gb2, gln_g, gln_b,
                                       gW3, gb3)],
        out_specs=[pl.BlockSpec((1, TB, 1), out3),
                   pl.BlockSpec((1, TB, 1), out3)],
        out_shape=[jax.ShapeDtypeStruct((T // TB, TB, 1), jnp.int32),
                   jax.ShapeDtypeStruct((T // TB, TB, 1), jnp.float32)],
    )(x, latent, gW1, gb1, gW2, gb2, gln_g, gln_b, gW3, gb3)
    return eidx.reshape(T), gval.reshape(T)


def _route_body(eidx_h, gval_h, xl_h,
                xls_h, gs_h, slots_h, bexp_h,
                ids_v, slots_v, cnt_v, cnt_all_v, off_s, allslots_v,
                perm_v, bexp_v, gval_v, gs_v, rows0_v, rows1_v,
                cnt_sh, slots_sh, sem, sem2):
    wid = lax.axis_index("s")
    tbase = wid * TPW
    sbase = wid * SPW

    pltpu.sync_copy(eidx_h.at[pl.ds(tbase, TPW)], ids_v)
    lane = lax.iota(jnp.int32, L)
    cnt = jnp.zeros((L,), jnp.int32)
    for k in range(TPW // L):
        v = ids_v[pl.ds(k * L, L)]
        for e in range(E):
            pc = plsc.all_reduce_population_count(v == e)
            cnt = cnt + jnp.where(lane == e, pc, 0)
    cnt_v[...] = cnt
    pltpu.sync_copy(cnt_v, cnt_sh.at[wid])
    plsc.subcore_barrier()

    pltpu.sync_copy(cnt_sh, cnt_all_v)
    total = jnp.zeros((L,), jnp.int32)
    mybase = jnp.zeros((L,), jnp.int32)
    for w in range(NS):
        row = cnt_all_v[w]
        total = total + row
        mybase = mybase + jnp.where(wid > w, row, 0)
    padded = ((total + (B - 1)) >> LOGB) << LOGB
    cum = plsc.cumsum(padded)
    astart = cum - padded
    off = astart + mybase
    for e in range(E):
        off_s[e] = off[e]

    for k in range(TPW // L):
        v = ids_v[pl.ds(k * L, L)]
        sv = jnp.zeros((L,), jnp.int32)
        for j in range(L):
            e = v[j]
            o = off_s[e]
            off_s[e] = o + 1
            sv = jnp.where(lane == j, o, sv)
        slots_v[pl.ds(k * L, L)] = sv
    pltpu.sync_copy(slots_v, slots_h.at[pl.ds(tbase, TPW)])
    pltpu.sync_copy(slots_v, slots_sh.at[pl.ds(tbase, TPW)])
    plsc.subcore_barrier()

    for k in range(NBP // L):
        bvec = (lane + k * L) << LOGB
        acc = jnp.zeros((L,), jnp.int32)
        for e in range(E):
            acc = acc + jnp.where(bvec >= cum[e], 1, 0)
        enc = jnp.where(bvec < cum[E - 1],
                        jnp.minimum(acc, E - 1) + E, E - 1)
        bexp_v[pl.ds(k * L, L)] = enc
    @pl.when(wid == 0)
    def _():
        pltpu.sync_copy(bexp_v, bexp_h)

    pltpu.sync_copy(slots_sh, allslots_v)
    for k in range(SPW // L):
        perm_v[pl.ds(k * L, L)] = jnp.zeros((L,), jnp.int32)
    def scat(k, _):
        s = allslots_v[pl.ds(k * L, L)]
        toks = k * L + lane
        mask = (s >= sbase) & (s < sbase + SPW)
        plsc.store_scatter(perm_v, [s - sbase], toks, mask=mask)
        return 0
    lax.fori_loop(0, T // L, scat, 0)

    pltpu.sync_copy(gval_h, gval_v)
    for k in range(SPW // L):
        pv = perm_v[pl.ds(k * L, L)]
        gs_v[pl.ds(k * L, L)] = plsc.load_gather(gval_v, [pv])
    pltpu.sync_copy(gs_v, gs_h.at[pl.ds(sbase, SPW)])

    nq = SPW // QR
    bufs = (rows0_v, rows1_v)
    sems = (sem, sem2)
    cps = [pltpu.async_copy(xl_h.at[perm_v.at[pl.ds(0, QR)]],
                            bufs[0], sems[0])]
    for q in range(nq):
        if q + 1 < nq:
            cps.append(pltpu.async_copy(
                xl_h.at[perm_v.at[pl.ds((q + 1) * QR, QR)]],
                bufs[(q + 1) % 2], sems[(q + 1) % 2]))
        cps[q].wait()
        pltpu.sync_copy(bufs[q % 2], xls_h.at[pl.ds(sbase + q * QR, QR), :])


def _route(eidx, gval, xl):
    mesh = plsc.VectorSubcoreMesh(core_axis_name="c", subcore_axis_name="s",
                                  num_cores=1)
    f = pl.kernel(
        _route_body,
        compiler_params=pltpu.CompilerParams(needs_layout_passes=False),
        out_type=[jax.ShapeDtypeStruct((P, XL // 2), jnp.int32),
                  jax.ShapeDtypeStruct((P,), jnp.float32),
                  jax.ShapeDtypeStruct((T,), jnp.int32),
                  jax.ShapeDtypeStruct((NBP,), jnp.int32)],
        mesh=mesh,
        scratch_types=[
            pltpu.VMEM((TPW,), jnp.int32),
            pltpu.VMEM((TPW,), jnp.int32),
            pltpu.VMEM((L,), jnp.int32),
            pltpu.VMEM((NS, L), jnp.int32),
            pltpu.SMEM((E,), jnp.int32),
            pltpu.VMEM((T,), jnp.int32),
            pltpu.VMEM((SPW,), jnp.int32),
            pltpu.VMEM((NBP,), jnp.int32),
            pltpu.VMEM((T,), jnp.float32),
            pltpu.VMEM((SPW,), jnp.float32),
            pltpu.VMEM((QR, XL // 2), jnp.int32),
            pltpu.VMEM((QR, XL // 2), jnp.int32),
            pltpu.VMEM_SHARED((NS, L), jnp.int32),
            pltpu.VMEM_SHARED((T,), jnp.int32),
            pltpu.SemaphoreType.DMA,
            pltpu.SemaphoreType.DMA,
        ],
    )
    return f(eidx, gval, xl)


def _mlp_body(bexp_sm, xls_ref, gs_ref, eW0, eb0, eWh, ebh, eWs, ebs,
              eWo, ebo, out_ref):
    enc = bexp_sm[pl.program_id(0)]
    @pl.when(enc >= E)
    def _():
        e = enc - E
        xb = xls_ref[:, :D_ENC]
        latb = xls_ref[:, D_ENC:]
        sel = (jax.lax.broadcasted_iota(jnp.int32, (D_LAT, CH), 0)
               == e * CH + jax.lax.broadcasted_iota(jnp.int32, (D_LAT, CH), 1)
               ).astype(jnp.bfloat16)
        chunk = _dot(latb, sel).astype(jnp.bfloat16)
        h0 = jnp.concatenate([xb, chunk], axis=-1)
        h = jax.nn.relu(_dot(h0, eW0[0]) + eb0[0]).astype(jnp.bfloat16)
        hidx = 0
        for i in range(1, DEPTH):
            if i == SKIP:
                h = jnp.concatenate([h, h0], axis=-1)
                h = jax.nn.relu(_dot(h, eWs[0]) + ebs[0])
            else:
                h = jax.nn.relu(_dot(h, eWh[0, hidx]) + ebh[0, hidx])
                hidx += 1
            h = h.astype(jnp.bfloat16)
        o = _dot(h, eWo[0]) + ebo[0]
        c = jnp.exp(o) * gs_ref[...]
        c = jnp.where(c == 0, _EPS, c)
        res = jnp.log(c)
        out_ref[...] = jnp.concatenate(
            [res, jnp.zeros((B, FW - OUT), jnp.float32)], axis=-1)


def _grouped_mlp(bexp, xls, gs, eW0, eb0, eWh, ebh, eWs, ebs, eWo, ebo):
    def ws(a):
        nd = a.ndim - 1
        return pl.BlockSpec((1,) + a.shape[1:],
                            lambda i, be, _n=nd: (be[i] & (E - 1),)
                            + (0,) * _n)
    grid_spec = pltpu.PrefetchScalarGridSpec(
        num_scalar_prefetch=1,
        grid=(NB,),
        in_specs=[
            pl.BlockSpec((B, XL), lambda i, be: (i, 0)),
            pl.BlockSpec((B, 1), lambda i, be: (i, 0)),
            ws(eW0), ws(eb0), ws(eWh), ws(ebh),
            ws(eWs), ws(ebs), ws(eWo), ws(ebo),
        ],
        out_specs=pl.BlockSpec((B, FW), lambda i, be: (i, 0)),
    )
    return pl.pallas_call(
        _mlp_body,
        grid_spec=grid_spec,
        out_shape=jax.ShapeDtypeStruct((P, FW), jnp.float32),
    )(bexp, xls, gs, eW0, eb0, eWh, ebh, eWs, ebs, eWo, ebo)


def _combine_body(fs_h, slots_h, out_h, myslots_v, rows_v, sem):
    wid = lax.axis_index("s")
    tbase = wid * TPW
    pltpu.sync_copy(slots_h.at[pl.ds(tbase, TPW)], myslots_v)
    pltpu.async_copy(fs_h.at[myslots_v], rows_v, sem).wait()
    pltpu.sync_copy(rows_v, out_h.at[pl.ds(tbase, TPW), :])


def _combine(fs, slots):
    mesh = plsc.VectorSubcoreMesh(core_axis_name="c", subcore_axis_name="s",
                                  num_cores=1)
    f = pl.kernel(
        _combine_body,
        compiler_params=pltpu.CompilerParams(needs_layout_passes=False),
        out_type=jax.ShapeDtypeStruct((T, FW), jnp.float32),
        mesh=mesh,
        scratch_types=[
            pltpu.VMEM((TPW,), jnp.int32),
            pltpu.VMEM((TPW, FW), jnp.float32),
            pltpu.SemaphoreType.DMA,
        ],
    )
    return f(fs, slots)


def kernel(x, latent, gW1, gb1, gW2, gb2, gln_g, gln_b, gW3, gb3,
           eW0, eb0, eWh, ebh, eWs, ebs, eWo, ebo):
    eidx, gval = _gate(x, latent, gW1, gb1, gW2, gb2, gln_g, gln_b, gW3, gb3)
    xlb = jnp.concatenate([x, latent], axis=1).astype(jnp.bfloat16)
    xl2 = jax.lax.bitcast_convert_type(xlb.reshape(T, XL // 2, 2), jnp.int32)
    xls2, gs, slots, bexp = _route(eidx, gval, xl2)
    xls = jax.lax.bitcast_convert_type(
        xls2, jnp.bfloat16).reshape(P, XL)
    bf = jnp.bfloat16
    fs = _grouped_mlp(bexp, xls, gs.reshape(P, 1),
                      eW0.astype(bf), eb0.reshape(E, 1, HID),
                      eWh.astype(bf), ebh, eWs.astype(bf),
                      ebs.reshape(E, 1, HID), eWo.astype(bf),
                      ebo.reshape(E, 1, OUT))
    out = _combine(fs, slots)
    return out[:, :OUT]


# speedup vs baseline: 2.4012x; 2.4012x over previous
import jax
import jax.numpy as jnp
import numpy as np
from jax import lax
from jax.experimental import pallas as pl
from jax.experimental.pallas import tpu as pltpu
from jax.experimental.pallas import tpu_sc as plsc

E = 8
K = 4
D_ENC = 256
D_LAT = 256
HID = 256
OUT = 4
DEPTH = 7
SKIP = 5
CH = D_LAT // E
XL = D_ENC + D_LAT

T = 4096
TB = 512
B = 256
LOGB = 8
NB = T // B + E
NBP = 32
P = NB * B
FW = 128

NC = 2
NS = 16
NW = NC * NS
TPW = T // NW
SR = 64
L = 16

_EPS = np.float32(np.finfo(np.float32).eps)
_LOG_EPS = np.float32(np.log(np.finfo(np.float32).eps))


def _dot(a, b):
    return jnp.dot(a, b, preferred_element_type=jnp.float32)


def _vgather(vec, idx):
    return lax.gather(
        vec, idx[:, None],
        lax.GatherDimensionNumbers(offset_dims=(),
                                   collapsed_slice_dims=(0,),
                                   start_index_map=(0,)),
        slice_sizes=(1,),
        mode=lax.GatherScatterMode.PROMISE_IN_BOUNDS)


def _gate_body(x_ref, lat_ref, gW1, gb1, gW2, gb2, gln_g, gln_b, gW3, gb3,
               tri_ref, eidx_ref, gval_ref, lgval_ref, rank_ref, cnt_ref,
               bexp_ref, run_ref):
    i = pl.program_id(0)
    @pl.when(i == 0)
    def _():
        run_ref[...] = jnp.zeros((1, E), jnp.float32)
    xb = x_ref[...]
    lb = lat_ref[...]
    g = jax.nn.relu(_dot(xb, gW1[:D_ENC]) + _dot(lb, gW1[D_ENC:]) + gb1[...])
    g = _dot(g, gW2[...]) + gb2[...]
    m = g.mean(-1, keepdims=True)
    v = ((g - m) ** 2).mean(-1, keepdims=True)
    g = (g - m) / jnp.sqrt(v + 1e-5) * gln_g[...] + gln_b[...]
    logits = _dot(g, gW3[...]) + gb3[...]
    mx = logits.max(-1, keepdims=True)
    s = jnp.exp(logits - mx)
    s = s / s.sum(-1, keepdims=True)
    lanes = jax.lax.broadcasted_iota(jnp.int32, s.shape, 1)
    cur = s
    sum4 = jnp.zeros((s.shape[0], 1), jnp.float32)
    eidx = gmax = None
    for r in range(K):
        m_r = cur.max(-1, keepdims=True)
        i_r = jnp.where(cur == m_r, lanes, E).min(-1, keepdims=True)
        sum4 = sum4 + m_r
        if r == 0:
            eidx, gmax = i_r, m_r
        cur = jnp.where(lanes == i_r, -jnp.inf, cur)
    gval = gmax / (sum4 + 1e-9)

    onehot = (lanes == eidx).astype(jnp.float32)
    prefix = _dot(tri_ref[...], onehot)
    run = run_ref[...]
    rank = ((prefix + run) * onehot).sum(-1, keepdims=True)
    newrun = run + onehot.sum(0, keepdims=True)
    run_ref[...] = newrun

    eidx_ref[...] = eidx.reshape(1, TB, 1)
    gval_ref[...] = gval.reshape(1, TB, 1)
    lgval_ref[...] = jnp.log(gval).reshape(1, TB, 1)
    rank_ref[...] = rank.astype(jnp.int32).reshape(1, TB, 1)

    @pl.when(i == pl.num_programs(0) - 1)
    def _():
        cnt = newrun.astype(jnp.int32)
        padded = ((cnt + (B - 1)) >> LOGB) << LOGB
        cum = padded
        for sh in (1, 2, 4):
            cum = cum + jnp.concatenate(
                [jnp.zeros((1, sh), jnp.int32), cum[:, :-sh]], axis=-1)
        astart = cum - padded
        cnt_ref[...] = jnp.concatenate(
            [astart, jnp.zeros((1, NBP - E), jnp.int32)], axis=-1)
        bstart = jax.lax.broadcasted_iota(jnp.int32, (1, NBP), 1) << LOGB
        acc = jnp.zeros((1, NBP), jnp.int32)
        for e in range(E):
            acc = acc + jnp.where(bstart >= cum[0, e], 1, 0)
        enc = jnp.where(bstart < cum[0, E - 1],
                        jnp.minimum(acc, E - 1) + E, E - 1)
        bexp_ref[...] = enc


def _gate(x, latent, gW1, gb1, gW2, gb2, gln_g, gln_b, gW3, gb3, tri):
    tok = lambda i: (i, 0)
    out3 = lambda i: (i, 0, 0)
    def wspec(a):
        return pl.BlockSpec(a.shape, lambda i, _a=a: tuple([0] * _a.ndim))
    res = pl.pallas_call(
        _gate_body,
        grid=(T // TB,),
        in_specs=[pl.BlockSpec((TB, D_ENC), tok),
                  pl.BlockSpec((TB, D_LAT), tok)]
                 + [wspec(a) for a in (gW1, gb1, gW2, gb2, gln_g, gln_b,
                                       gW3, gb3)]
                 + [pl.BlockSpec((TB, TB), lambda i: (0, 0))],
        out_specs=[pl.BlockSpec((1, TB, 1), out3),
                   pl.BlockSpec((1, TB, 1), out3),
                   pl.BlockSpec((1, TB, 1), out3),
                   pl.BlockSpec((1, TB, 1), out3),
                   pl.BlockSpec((1, NBP), lambda i: (0, 0)),
                   pl.BlockSpec((1, NBP), lambda i: (0, 0))],
        out_shape=[jax.ShapeDtypeStruct((T // TB, TB, 1), jnp.int32),
                   jax.ShapeDtypeStruct((T // TB, TB, 1), jnp.float32),
                   jax.ShapeDtypeStruct((T // TB, TB, 1), jnp.float32),
                   jax.ShapeDtypeStruct((T // TB, TB, 1), jnp.int32),
                   jax.ShapeDtypeStruct((1, NBP), jnp.int32),
                   jax.ShapeDtypeStruct((1, NBP), jnp.int32)],
        scratch_shapes=[pltpu.VMEM((1, E), jnp.float32)],
    )(x, latent, gW1, gb1, gW2, gb2, gln_g, gln_b, gW3, gb3, tri)
    eidx, gval, lgval, rank, astart, bexp = res
    return (eidx.reshape(T), gval.reshape(T), lgval.reshape(T),
            rank.reshape(T), astart.reshape(NBP), bexp.reshape(NBP))


def _dispatch_body(eidx_h, rank_h, astart_h, xl_h,
                   xls_h, slots_h,
                   ids_v, rank_v, astart_v, slots_lin_v, rows_v,
                   sem):
    wid = lax.axis_index("s") * NC + lax.axis_index("c")
    tbase = wid * TPW

    pltpu.sync_copy(eidx_h.at[pl.ds(tbase, TPW)], ids_v)
    pltpu.sync_copy(rank_h.at[pl.ds(tbase, TPW)], rank_v)
    pltpu.sync_copy(astart_h.at[pl.ds(0, L)], astart_v)
    astart = astart_v[pl.ds(0, L)]
    for k in range(TPW // L):
        v = ids_v[pl.ds(k * L, L)]
        base = _vgather(astart, v)
        slot = base + rank_v[pl.ds(k * L, L)]
        slots_lin_v[pl.ds(k * L, L)] = slot
    pltpu.sync_copy(slots_lin_v, slots_h.at[pl.ds(tbase, TPW)])

    pltpu.sync_copy(xl_h.at[pl.ds(tbase, TPW)], rows_v)
    pltpu.async_copy(rows_v, xls_h.at[slots_lin_v], sem).wait()


def _dispatch(eidx, rank, astart, xl):
    mesh = plsc.VectorSubcoreMesh(core_axis_name="c", subcore_axis_name="s")
    f = pl.kernel(
        _dispatch_body,
        compiler_params=pltpu.CompilerParams(needs_layout_passes=False),
        out_type=[jax.ShapeDtypeStruct((P, 1, XL), jnp.float32),
                  jax.ShapeDtypeStruct((T,), jnp.int32)],
        mesh=mesh,
        scratch_types=[
            pltpu.VMEM((TPW,), jnp.int32),
            pltpu.VMEM((TPW,), jnp.int32),
            pltpu.VMEM((L,), jnp.int32),
            pltpu.VMEM((TPW,), jnp.int32),
            pltpu.VMEM((TPW, 1, XL), jnp.float32),
            pltpu.SemaphoreType.DMA,
        ],
    )
    return f(eidx, rank, astart, xl)


def _mlp_body(bexp_sm, xls_ref, eW0, eb0, eWh, ebh, eWs, ebs,
              eWo, ebo, out_ref):
    enc = bexp_sm[pl.program_id(0)]
    @pl.when(enc >= E)
    def _():
        e = enc - E
        xls = xls_ref[...].astype(jnp.bfloat16)
        xb = xls[:, :D_ENC]
        latb = xls[:, D_ENC:]
        sel = (jax.lax.broadcasted_iota(jnp.int32, (D_LAT, CH), 0)
               == e * CH + jax.lax.broadcasted_iota(jnp.int32, (D_LAT, CH), 1)
               ).astype(jnp.bfloat16)
        chunk = _dot(latb, sel).astype(jnp.bfloat16)
        h0 = jnp.concatenate([xb, chunk], axis=-1)
        h = jax.nn.relu(_dot(h0, eW0[0]) + eb0[0]).astype(jnp.bfloat16)
        hidx = 0
        for i in range(1, DEPTH):
            if i == SKIP:
                h = jnp.concatenate([h, h0], axis=-1)
                h = jax.nn.relu(_dot(h, eWs[0]) + ebs[0])
            else:
                h = jax.nn.relu(_dot(h, eWh[0, hidx]) + ebh[0, hidx])
                hidx += 1
            h = h.astype(jnp.bfloat16)
        o = _dot(h, eWo[0]) + ebo[0]
        out_ref[...] = jnp.concatenate(
            [o, jnp.zeros((B, FW - OUT), jnp.float32)], axis=-1)


def _grouped_mlp(bexp, xls, eW0, eb0, eWh, ebh, eWs, ebs, eWo, ebo):
    def ws(a):
        nd = a.ndim - 1
        return pl.BlockSpec((1,) + a.shape[1:],
                            lambda i, be, _n=nd: (be[i] & (E - 1),)
                            + (0,) * _n)
    grid_spec = pltpu.PrefetchScalarGridSpec(
        num_scalar_prefetch=1,
        grid=(NB,),
        in_specs=[
            pl.BlockSpec((B, XL), lambda i, be: (i, 0)),
            ws(eW0), ws(eb0), ws(eWh), ws(ebh),
            ws(eWs), ws(ebs), ws(eWo), ws(ebo),
        ],
        out_specs=pl.BlockSpec((B, FW), lambda i, be: (i, 0)),
    )
    return pl.pallas_call(
        _mlp_body,
        grid_spec=grid_spec,
        out_shape=jax.ShapeDtypeStruct((P, FW), jnp.float32),
    )(bexp, xls, eW0, eb0, eWh, ebh, eWs, ebs, eWo, ebo)


def _combine_body(fs_h, slots_h, gval_h, lgval_h, out_h,
                  myslots_v, gval_v, lgval_v, rows_v, out_v, sem):
    wid = lax.axis_index("s") * NC + lax.axis_index("c")
    tbase = wid * TPW
    pltpu.sync_copy(slots_h.at[pl.ds(tbase, TPW)], myslots_v)
    pltpu.sync_copy(gval_h.at[pl.ds(tbase, TPW)], gval_v)
    pltpu.sync_copy(lgval_h.at[pl.ds(tbase, TPW)], lgval_v)
    pltpu.async_copy(fs_h.at[myslots_v], rows_v, sem).wait()
    lane = lax.iota(jnp.int32, L)
    for k in range(TPW // L):
        tloc = k * L + lane
        g16 = gval_v[pl.ds(k * L, L)]
        lg16 = lgval_v[pl.ds(k * L, L)]
        for j in range(OUT):
            val = plsc.load_gather(rows_v, [tloc, lane * 0 + j])
            c = jnp.exp(val) * g16
            res = jnp.where(c == 0, _LOG_EPS, val + lg16)
            plsc.store_scatter(out_v, [tloc * OUT + j], res)
    pltpu.sync_copy(out_v, out_h.at[pl.ds(tbase * OUT, TPW * OUT)])


def _combine(fs, slots, gval, lgval):
    mesh = plsc.VectorSubcoreMesh(core_axis_name="c", subcore_axis_name="s")
    f = pl.kernel(
        _combine_body,
        compiler_params=pltpu.CompilerParams(needs_layout_passes=False),
        out_type=jax.ShapeDtypeStruct((T * OUT,), jnp.float32),
        mesh=mesh,
        scratch_types=[
            pltpu.VMEM((TPW,), jnp.int32),
            pltpu.VMEM((TPW,), jnp.float32),
            pltpu.VMEM((TPW,), jnp.float32),
            pltpu.VMEM((TPW, FW), jnp.float32),
            pltpu.VMEM((TPW * OUT,), jnp.float32),
            pltpu.SemaphoreType.DMA,
        ],
    )
    return f(fs, slots, gval, lgval)


def kernel(x, latent, gW1, gb1, gW2, gb2, gln_g, gln_b, gW3, gb3,
           eW0, eb0, eWh, ebh, eWs, ebs, eWo, ebo):
    r = jnp.arange(TB, dtype=jnp.int32)
    tri = (r[:, None] > r[None, :]).astype(jnp.float32)
    eidx, gval, lgval, rank, astart, bexp = _gate(
        x, latent, gW1, gb1, gW2, gb2, gln_g, gln_b, gW3, gb3, tri)
    xl = jnp.concatenate([x, latent], axis=1).reshape(T, 1, XL)
    xls3, slots = _dispatch(eidx, rank, astart, xl)
    xls = xls3.reshape(P, XL)
    bf = jnp.bfloat16
    fs = _grouped_mlp(bexp, xls,
                      eW0.astype(bf), eb0.reshape(E, 1, HID),
                      eWh.astype(bf), ebh, eWs.astype(bf),
                      ebs.reshape(E, 1, HID), eWo.astype(bf),
                      ebo.reshape(E, 1, OUT))
    out = _combine(fs, slots, gval, lgval)
    return out.reshape(T, OUT)


# speedup vs baseline: 3.4067x; 1.4187x over previous
import jax
import jax.numpy as jnp
import numpy as np
from jax import lax
from jax.experimental import pallas as pl
from jax.experimental.pallas import tpu as pltpu
from jax.experimental.pallas import tpu_sc as plsc

E = 8
K = 4
D_ENC = 256
D_LAT = 256
HID = 256
OUT = 4
DEPTH = 7
SKIP = 5
CH = D_LAT // E
XL = D_ENC + D_LAT

T = 4096
TB = 512
B = 256
LOGB = 8
NB = T // B + E
NBP = 32
P = NB * B
FW = 128

NC = 2
NS = 16
NW = NC * NS
TPW = T // NW
SR = 64
L = 16

_EPS = np.float32(np.finfo(np.float32).eps)
_LOG_EPS = np.float32(np.log(np.finfo(np.float32).eps))


def _dot(a, b):
    return jnp.dot(a, b, preferred_element_type=jnp.float32)


def _vgather(vec, idx):
    return lax.gather(
        vec, idx[:, None],
        lax.GatherDimensionNumbers(offset_dims=(),
                                   collapsed_slice_dims=(0,),
                                   start_index_map=(0,)),
        slice_sizes=(1,),
        mode=lax.GatherScatterMode.PROMISE_IN_BOUNDS)


def _gate_body(x_ref, lat_ref, gW1, gb1, gW2, gb2, gln_g, gln_b, gW3, gb3,
               tri_ref, eidx_ref, gval_ref, lgval_ref, rank_ref, cnt_ref,
               bexp_ref, run_ref):
    i = pl.program_id(0)
    @pl.when(i == 0)
    def _():
        run_ref[...] = jnp.zeros((1, E), jnp.float32)
    xb = x_ref[...]
    lb = lat_ref[...]
    g = jax.nn.relu(_dot(xb, gW1[:D_ENC]) + _dot(lb, gW1[D_ENC:]) + gb1[...])
    g = _dot(g, gW2[...]) + gb2[...]
    m = g.mean(-1, keepdims=True)
    v = ((g - m) ** 2).mean(-1, keepdims=True)
    g = (g - m) / jnp.sqrt(v + 1e-5) * gln_g[...] + gln_b[...]
    logits = _dot(g, gW3[...]) + gb3[...]
    mx = logits.max(-1, keepdims=True)
    s = jnp.exp(logits - mx)
    s = s / s.sum(-1, keepdims=True)
    lanes = jax.lax.broadcasted_iota(jnp.int32, s.shape, 1)
    cur = s
    sum4 = jnp.zeros((s.shape[0], 1), jnp.float32)
    eidx = gmax = None
    for r in range(K):
        m_r = cur.max(-1, keepdims=True)
        i_r = jnp.where(cur == m_r, lanes, E).min(-1, keepdims=True)
        sum4 = sum4 + m_r
        if r == 0:
            eidx, gmax = i_r, m_r
        cur = jnp.where(lanes == i_r, -jnp.inf, cur)
    gval = gmax / (sum4 + 1e-9)

    onehot = (lanes == eidx).astype(jnp.float32)
    prefix = _dot(tri_ref[...], onehot)
    run = run_ref[...]
    rank = ((prefix + run) * onehot).sum(-1, keepdims=True)
    newrun = run + onehot.sum(0, keepdims=True)
    run_ref[...] = newrun

    eidx_ref[...] = eidx.reshape(1, TB, 1)
    gval_ref[...] = gval.reshape(1, TB, 1)
    lgval_ref[...] = jnp.log(gval).reshape(1, TB, 1)
    rank_ref[...] = rank.astype(jnp.int32).reshape(1, TB, 1)

    @pl.when(i == pl.num_programs(0) - 1)
    def _():
        cnt = newrun.astype(jnp.int32)
        padded = ((cnt + (B - 1)) >> LOGB) << LOGB
        cum = padded
        for sh in (1, 2, 4):
            cum = cum + jnp.concatenate(
                [jnp.zeros((1, sh), jnp.int32), cum[:, :-sh]], axis=-1)
        astart = cum - padded
        cnt_ref[...] = jnp.concatenate(
            [astart, jnp.zeros((1, NBP - E), jnp.int32)], axis=-1)
        bstart = jax.lax.broadcasted_iota(jnp.int32, (1, NBP), 1) << LOGB
        acc = jnp.zeros((1, NBP), jnp.int32)
        for e in range(E):
            acc = acc + jnp.where(bstart >= cum[0, e], 1, 0)
        enc = jnp.where(bstart < cum[0, E - 1],
                        jnp.minimum(acc, E - 1) + E, E - 1)
        bexp_ref[...] = enc


def _gate(x, latent, gW1, gb1, gW2, gb2, gln_g, gln_b, gW3, gb3, tri):
    tok = lambda i: (i, 0)
    out3 = lambda i: (i, 0, 0)
    def wspec(a):
        return pl.BlockSpec(a.shape, lambda i, _a=a: tuple([0] * _a.ndim))
    res = pl.pallas_call(
        _gate_body,
        grid=(T // TB,),
        in_specs=[pl.BlockSpec((TB, D_ENC), tok),
                  pl.BlockSpec((TB, D_LAT), tok)]
                 + [wspec(a) for a in (gW1, gb1, gW2, gb2, gln_g, gln_b,
                                       gW3, gb3)]
                 + [pl.BlockSpec((TB, TB), lambda i: (0, 0))],
        out_specs=[pl.BlockSpec((1, TB, 1), out3),
                   pl.BlockSpec((1, TB, 1), out3),
                   pl.BlockSpec((1, TB, 1), out3),
                   pl.BlockSpec((1, TB, 1), out3),
                   pl.BlockSpec((1, NBP), lambda i: (0, 0)),
                   pl.BlockSpec((1, NBP), lambda i: (0, 0))],
        out_shape=[jax.ShapeDtypeStruct((T // TB, TB, 1), jnp.int32),
                   jax.ShapeDtypeStruct((T // TB, TB, 1), jnp.float32),
                   jax.ShapeDtypeStruct((T // TB, TB, 1), jnp.float32),
                   jax.ShapeDtypeStruct((T // TB, TB, 1), jnp.int32),
                   jax.ShapeDtypeStruct((1, NBP), jnp.int32),
                   jax.ShapeDtypeStruct((1, NBP), jnp.int32)],
        scratch_shapes=[pltpu.VMEM((1, E), jnp.float32)],
    )(x, latent, gW1, gb1, gW2, gb2, gln_g, gln_b, gW3, gb3, tri)
    eidx, gval, lgval, rank, astart, bexp = res
    return (eidx.reshape(T), gval.reshape(T), lgval.reshape(T),
            rank.reshape(T), astart.reshape(NBP), bexp.reshape(NBP))


def _dispatch_body(eidx_h, rank_h, astart_h, x_h, lat_h,
                   xls_h, slots_h,
                   ids_v, rank_v, astart_v, slots_lin_v, rows_v,
                   sem, sem2):
    wid = lax.axis_index("s") * NC + lax.axis_index("c")
    tbase = wid * TPW

    cpx = pltpu.async_copy(x_h.at[pl.ds(tbase, TPW)],
                           rows_v.at[:, 0, pl.ds(0, D_ENC)], sem)
    cpl = pltpu.async_copy(lat_h.at[pl.ds(tbase, TPW)],
                           rows_v.at[:, 0, pl.ds(D_ENC, D_LAT)], sem2)
    pltpu.sync_copy(eidx_h.at[pl.ds(tbase, TPW)], ids_v)
    pltpu.sync_copy(rank_h.at[pl.ds(tbase, TPW)], rank_v)
    pltpu.sync_copy(astart_h.at[pl.ds(0, L)], astart_v)
    astart = astart_v[pl.ds(0, L)]
    for k in range(TPW // L):
        v = ids_v[pl.ds(k * L, L)]
        base = _vgather(astart, v)
        slot = base + rank_v[pl.ds(k * L, L)]
        slots_lin_v[pl.ds(k * L, L)] = slot
    pltpu.sync_copy(slots_lin_v, slots_h.at[pl.ds(tbase, TPW)])

    cpx.wait()
    cpl.wait()
    pltpu.async_copy(rows_v, xls_h.at[slots_lin_v], sem).wait()


def _dispatch(eidx, rank, astart, x, latent):
    mesh = plsc.VectorSubcoreMesh(core_axis_name="c", subcore_axis_name="s")
    f = pl.kernel(
        _dispatch_body,
        compiler_params=pltpu.CompilerParams(needs_layout_passes=False),
        out_type=[jax.ShapeDtypeStruct((P, 1, XL), jnp.float32),
                  jax.ShapeDtypeStruct((T,), jnp.int32)],
        mesh=mesh,
        scratch_types=[
            pltpu.VMEM((TPW,), jnp.int32),
            pltpu.VMEM((TPW,), jnp.int32),
            pltpu.VMEM((L,), jnp.int32),
            pltpu.VMEM((TPW,), jnp.int32),
            pltpu.VMEM((TPW, 1, XL), jnp.float32),
            pltpu.SemaphoreType.DMA,
            pltpu.SemaphoreType.DMA,
        ],
    )
    return f(eidx, rank, astart, x, latent)


def _mlp_body(bexp_sm, xls_ref, eW0, eb0, eWh, ebh, eWs, ebs,
              eWo, ebo, out_ref):
    enc = bexp_sm[pl.program_id(0)]
    @pl.when(enc >= E)
    def _():
        e = enc - E
        xls = xls_ref[:, 0, :]
        xb = xls[:, :D_ENC]
        latb = xls[:, D_ENC:]
        sel = (jax.lax.broadcasted_iota(jnp.int32, (D_LAT, CH), 0)
               == e * CH + jax.lax.broadcasted_iota(jnp.int32, (D_LAT, CH), 1)
               ).astype(jnp.float32)
        chunk = _dot(latb, sel)
        h0 = jnp.concatenate([xb, chunk], axis=-1)
        h = jax.nn.relu(_dot(h0, eW0[0]) + eb0[0])
        hidx = 0
        for i in range(1, DEPTH):
            if i == SKIP:
                h = jnp.concatenate([h, h0], axis=-1)
                h = jax.nn.relu(_dot(h, eWs[0]) + ebs[0])
            else:
                h = jax.nn.relu(_dot(h, eWh[0, hidx]) + ebh[0, hidx])
                hidx += 1
        o = _dot(h, eWo[0]) + ebo[0]
        out_ref[...] = jnp.concatenate(
            [o, jnp.zeros((B, FW - OUT), jnp.float32)], axis=-1)


def _grouped_mlp(bexp, xls, eW0, eb0, eWh, ebh, eWs, ebs, eWo, ebo):
    def ws(a):
        nd = a.ndim - 1
        return pl.BlockSpec((1,) + a.shape[1:],
                            lambda i, be, _n=nd: (be[i] & (E - 1),)
                            + (0,) * _n)
    grid_spec = pltpu.PrefetchScalarGridSpec(
        num_scalar_prefetch=1,
        grid=(NB,),
        in_specs=[
            pl.BlockSpec((B, 1, XL), lambda i, be: (i, 0, 0)),
            ws(eW0), ws(eb0), ws(eWh), ws(ebh),
            ws(eWs), ws(ebs), ws(eWo), ws(ebo),
        ],
        out_specs=pl.BlockSpec((B, FW), lambda i, be: (i, 0)),
    )
    return pl.pallas_call(
        _mlp_body,
        grid_spec=grid_spec,
        out_shape=jax.ShapeDtypeStruct((P, FW), jnp.float32),
    )(bexp, xls, eW0, eb0, eWh, ebh, eWs, ebs, eWo, ebo)


def _combine_body(fs_h, slots_h, gval_h, lgval_h, out_h,
                  myslots_v, gval_v, lgval_v, rows_v, out_v, sem):
    wid = lax.axis_index("s") * NC + lax.axis_index("c")
    tbase = wid * TPW
    pltpu.sync_copy(slots_h.at[pl.ds(tbase, TPW)], myslots_v)
    pltpu.sync_copy(gval_h.at[pl.ds(tbase, TPW)], gval_v)
    pltpu.sync_copy(lgval_h.at[pl.ds(tbase, TPW)], lgval_v)
    pltpu.async_copy(fs_h.at[myslots_v], rows_v, sem).wait()
    lane = lax.iota(jnp.int32, L)
    for k in range(TPW // L):
        tloc = k * L + lane
        g16 = gval_v[pl.ds(k * L, L)]
        lg16 = lgval_v[pl.ds(k * L, L)]
        for j in range(OUT):
            val = plsc.load_gather(rows_v, [tloc, lane * 0 + j])
            c = jnp.exp(val) * g16
            res = jnp.where(c == 0, _LOG_EPS, val + lg16)
            plsc.store_scatter(out_v, [tloc * OUT + j], res)
    pltpu.sync_copy(out_v, out_h.at[pl.ds(tbase * OUT, TPW * OUT)])


def _combine(fs, slots, gval, lgval):
    mesh = plsc.VectorSubcoreMesh(core_axis_name="c", subcore_axis_name="s")
    f = pl.kernel(
        _combine_body,
        compiler_params=pltpu.CompilerParams(needs_layout_passes=False),
        out_type=jax.ShapeDtypeStruct((T * OUT,), jnp.float32),
        mesh=mesh,
        scratch_types=[
            pltpu.VMEM((TPW,), jnp.int32),
            pltpu.VMEM((TPW,), jnp.float32),
            pltpu.VMEM((TPW,), jnp.float32),
            pltpu.VMEM((TPW, FW), jnp.float32),
            pltpu.VMEM((TPW * OUT,), jnp.float32),
            pltpu.SemaphoreType.DMA,
        ],
    )
    return f(fs, slots, gval, lgval)


_TRI = np.tril(np.ones((TB, TB), np.float32), -1)


def kernel(x, latent, gW1, gb1, gW2, gb2, gln_g, gln_b, gW3, gb3,
           eW0, eb0, eWh, ebh, eWs, ebs, eWo, ebo):
    tri = jnp.asarray(_TRI)
    eidx, gval, lgval, rank, astart, bexp = _gate(
        x, latent, gW1, gb1, gW2, gb2, gln_g, gln_b, gW3, gb3, tri)
    xls, slots = _dispatch(eidx, rank, astart, x, latent)
    fs = _grouped_mlp(bexp, xls,
                      eW0, eb0.reshape(E, 1, HID),
                      eWh, ebh, eWs,
                      ebs.reshape(E, 1, HID), eWo,
                      ebo.reshape(E, 1, OUT))
    out = _combine(fs, slots, gval, lgval)
    return out.reshape(T, OUT)


# speedup vs baseline: 3.4502x; 1.0128x over previous
import jax
import jax.numpy as jnp
import numpy as np
from jax import lax
from jax.experimental import pallas as pl
from jax.experimental.pallas import tpu as pltpu
from jax.experimental.pallas import tpu_sc as plsc

E = 8
K = 4
D_ENC = 256
D_LAT = 256
HID = 256
OUT = 4
DEPTH = 7
SKIP = 5
CH = D_LAT // E
XL = D_ENC + D_LAT

T = 4096
TB = 512
B = 256
LOGB = 8
NB = T // B + E
NBP = 32
P = NB * B
FW = 128

NC = 2
NS = 16
NW = NC * NS
TPW = T // NW
SR = 64
L = 16

_EPS = np.float32(np.finfo(np.float32).eps)
_LOG_EPS = np.float32(np.log(np.finfo(np.float32).eps))


def _dot(a, b):
    return jnp.dot(a, b, preferred_element_type=jnp.float32)


def _vgather(vec, idx):
    return lax.gather(
        vec, idx[:, None],
        lax.GatherDimensionNumbers(offset_dims=(),
                                   collapsed_slice_dims=(0,),
                                   start_index_map=(0,)),
        slice_sizes=(1,),
        mode=lax.GatherScatterMode.PROMISE_IN_BOUNDS)


def _gate_body(x_ref, lat_ref, gW1, gb1, gW2, gb2, gln_g, gln_b, gW3, gb3,
               tri_ref, eidx_ref, gval_ref, lgval_ref, rank_ref, cnt_ref,
               bexp_ref, run_ref):
    i = pl.program_id(0)
    @pl.when(i == 0)
    def _():
        run_ref[...] = jnp.zeros((1, E), jnp.float32)
    xb = x_ref[...]
    lb = lat_ref[...]
    g = jax.nn.relu(_dot(xb, gW1[:D_ENC]) + _dot(lb, gW1[D_ENC:]) + gb1[...])
    g = _dot(g, gW2[...]) + gb2[...]
    m = g.mean(-1, keepdims=True)
    v = ((g - m) ** 2).mean(-1, keepdims=True)
    g = (g - m) / jnp.sqrt(v + 1e-5) * gln_g[...] + gln_b[...]
    logits = _dot(g, gW3[...]) + gb3[...]
    mx = logits.max(-1, keepdims=True)
    s = jnp.exp(logits - mx)
    s = s / s.sum(-1, keepdims=True)
    lanes = jax.lax.broadcasted_iota(jnp.int32, s.shape, 1)
    cur = s
    sum4 = jnp.zeros((s.shape[0], 1), jnp.float32)
    eidx = gmax = None
    for r in range(K):
        m_r = cur.max(-1, keepdims=True)
        sum4 = sum4 + m_r
        if r == 0:
            eidx = jnp.where(cur == m_r, lanes, E).min(-1, keepdims=True)
            gmax = m_r
        cur = jnp.where(cur == m_r, -jnp.inf, cur)
    gval = gmax / (sum4 + 1e-9)

    onehot = (lanes == eidx).astype(jnp.float32)
    prefix = _dot(tri_ref[...], onehot)
    run = run_ref[...]
    rank = ((prefix + run) * onehot).sum(-1, keepdims=True)
    newrun = run + onehot.sum(0, keepdims=True)
    run_ref[...] = newrun

    eidx_ref[...] = eidx.reshape(1, TB, 1)
    gval_ref[...] = gval.reshape(1, TB, 1)
    lgval_ref[...] = jnp.log(gval).reshape(1, TB, 1)
    rank_ref[...] = rank.astype(jnp.int32).reshape(1, TB, 1)

    @pl.when(i == pl.num_programs(0) - 1)
    def _():
        cnt = newrun.astype(jnp.int32)
        padded = ((cnt + (B - 1)) >> LOGB) << LOGB
        cum = padded
        for sh in (1, 2, 4):
            cum = cum + jnp.concatenate(
                [jnp.zeros((1, sh), jnp.int32), cum[:, :-sh]], axis=-1)
        astart = cum - padded
        cnt_ref[...] = jnp.concatenate(
            [astart, jnp.zeros((1, NBP - E), jnp.int32)], axis=-1)
        bstart = jax.lax.broadcasted_iota(jnp.int32, (1, NBP), 1) << LOGB
        acc = jnp.zeros((1, NBP), jnp.int32)
        for e in range(E):
            acc = acc + jnp.where(bstart >= cum[0, e], 1, 0)
        enc = jnp.where(bstart < cum[0, E - 1],
                        jnp.minimum(acc, E - 1) + E, E - 1)
        bexp_ref[...] = enc


def _gate(x, latent, gW1, gb1, gW2, gb2, gln_g, gln_b, gW3, gb3, tri):
    tok = lambda i: (i, 0)
    out3 = lambda i: (i, 0, 0)
    def wspec(a):
        return pl.BlockSpec(a.shape, lambda i, _a=a: tuple([0] * _a.ndim))
    res = pl.pallas_call(
        _gate_body,
        grid=(T // TB,),
        in_specs=[pl.BlockSpec((TB, D_ENC), tok),
                  pl.BlockSpec((TB, D_LAT), tok)]
                 + [wspec(a) for a in (gW1, gb1, gW2, gb2, gln_g, gln_b,
                                       gW3, gb3)]
                 + [pl.BlockSpec((TB, TB), lambda i: (0, 0))],
        out_specs=[pl.BlockSpec((1, TB, 1), out3),
                   pl.BlockSpec((1, TB, 1), out3),
                   pl.BlockSpec((1, TB, 1), out3),
                   pl.BlockSpec((1, TB, 1), out3),
                   pl.BlockSpec((1, NBP), lambda i: (0, 0)),
                   pl.BlockSpec((1, NBP), lambda i: (0, 0))],
        out_shape=[jax.ShapeDtypeStruct((T // TB, TB, 1), jnp.int32),
                   jax.ShapeDtypeStruct((T // TB, TB, 1), jnp.float32),
                   jax.ShapeDtypeStruct((T // TB, TB, 1), jnp.float32),
                   jax.ShapeDtypeStruct((T // TB, TB, 1), jnp.int32),
                   jax.ShapeDtypeStruct((1, NBP), jnp.int32),
                   jax.ShapeDtypeStruct((1, NBP), jnp.int32)],
        scratch_shapes=[pltpu.VMEM((1, E), jnp.float32)],
    )(x, latent, gW1, gb1, gW2, gb2, gln_g, gln_b, gW3, gb3, tri)
    eidx, gval, lgval, rank, astart, bexp = res
    return (eidx.reshape(T), gval.reshape(T), lgval.reshape(T),
            rank.reshape(T), astart.reshape(NBP), bexp.reshape(NBP))


def _dispatch_body(eidx_h, rank_h, astart_h, x_h, lat_h,
                   xls_h, slots_h,
                   ids_v, rank_v, astart_v, slots_lin_v, rows_v,
                   sem, sem2):
    wid = lax.axis_index("s") * NC + lax.axis_index("c")
    tbase = wid * TPW

    cpx = pltpu.async_copy(x_h.at[pl.ds(tbase, TPW)],
                           rows_v.at[:, 0, pl.ds(0, D_ENC)], sem)
    cpl = pltpu.async_copy(lat_h.at[pl.ds(tbase, TPW)],
                           rows_v.at[:, 0, pl.ds(D_ENC, D_LAT)], sem2)
    pltpu.sync_copy(eidx_h.at[pl.ds(tbase, TPW)], ids_v)
    pltpu.sync_copy(rank_h.at[pl.ds(tbase, TPW)], rank_v)
    pltpu.sync_copy(astart_h.at[pl.ds(0, L)], astart_v)
    astart = astart_v[pl.ds(0, L)]
    for k in range(TPW // L):
        v = ids_v[pl.ds(k * L, L)]
        base = _vgather(astart, v)
        slot = base + rank_v[pl.ds(k * L, L)]
        slots_lin_v[pl.ds(k * L, L)] = slot
    pltpu.sync_copy(slots_lin_v, slots_h.at[pl.ds(tbase, TPW)])

    cpx.wait()
    cpl.wait()
    pltpu.async_copy(rows_v, xls_h.at[slots_lin_v], sem).wait()


def _dispatch(eidx, rank, astart, x, latent):
    mesh = plsc.VectorSubcoreMesh(core_axis_name="c", subcore_axis_name="s")
    f = pl.kernel(
        _dispatch_body,
        compiler_params=pltpu.CompilerParams(needs_layout_passes=False),
        out_type=[jax.ShapeDtypeStruct((P, 1, XL), jnp.float32),
                  jax.ShapeDtypeStruct((T,), jnp.int32)],
        mesh=mesh,
        scratch_types=[
            pltpu.VMEM((TPW,), jnp.int32),
            pltpu.VMEM((TPW,), jnp.int32),
            pltpu.VMEM((L,), jnp.int32),
            pltpu.VMEM((TPW,), jnp.int32),
            pltpu.VMEM((TPW, 1, XL), jnp.float32),
            pltpu.SemaphoreType.DMA,
            pltpu.SemaphoreType.DMA,
        ],
    )
    return f(eidx, rank, astart, x, latent)


def _mlp_body(bexp_sm, xls_ref, eW0, eb0, eWh, ebh, eWs, ebs,
              eWo, ebo, out_ref):
    enc = bexp_sm[pl.program_id(0)]
    @pl.when(enc >= E)
    def _():
        e = enc - E
        xls = xls_ref[:, 0, :]
        sel = (jax.lax.broadcasted_iota(jnp.int32, (D_LAT, CH), 0)
               == e * CH + jax.lax.broadcasted_iota(jnp.int32, (D_LAT, CH), 1)
               ).astype(jnp.float32)
        B2 = B // 2
        halves = [xls[:B2], xls[B2:]]
        chunks = [_dot(hh[:, D_ENC:], sel) for hh in halves]
        h0s = [jnp.concatenate([hh[:, :D_ENC], ck], axis=-1)
               for hh, ck in zip(halves, chunks)]
        hs = [jax.nn.relu(_dot(h0, eW0[0]) + eb0[0]) for h0 in h0s]
        hidx = 0
        for i in range(1, DEPTH):
            if i == SKIP:
                hs = [jnp.concatenate([h, h0], axis=-1)
                      for h, h0 in zip(hs, h0s)]
                hs = [jax.nn.relu(_dot(h, eWs[0]) + ebs[0]) for h in hs]
            else:
                hs = [jax.nn.relu(_dot(h, eWh[0, hidx]) + ebh[0, hidx])
                      for h in hs]
                hidx += 1
        os_ = [_dot(h, eWo[0]) + ebo[0] for h in hs]
        o = jnp.concatenate(os_, axis=0)
        out_ref[...] = jnp.concatenate(
            [o, jnp.zeros((B, FW - OUT), jnp.float32)], axis=-1)


def _grouped_mlp(bexp, xls, eW0, eb0, eWh, ebh, eWs, ebs, eWo, ebo):
    def ws(a):
        nd = a.ndim - 1
        return pl.BlockSpec((1,) + a.shape[1:],
                            lambda i, be, _n=nd: (be[i] & (E - 1),)
                            + (0,) * _n)
    grid_spec = pltpu.PrefetchScalarGridSpec(
        num_scalar_prefetch=1,
        grid=(NB,),
        in_specs=[
            pl.BlockSpec((B, 1, XL), lambda i, be: (i, 0, 0)),
            ws(eW0), ws(eb0), ws(eWh), ws(ebh),
            ws(eWs), ws(ebs), ws(eWo), ws(ebo),
        ],
        out_specs=pl.BlockSpec((B, FW), lambda i, be: (i, 0)),
    )
    return pl.pallas_call(
        _mlp_body,
        grid_spec=grid_spec,
        out_shape=jax.ShapeDtypeStruct((P, FW), jnp.float32),
    )(bexp, xls, eW0, eb0, eWh, ebh, eWs, ebs, eWo, ebo)


def _combine_body(fs_h, slots_h, gval_h, lgval_h, out_h,
                  myslots_v, gval_v, lgval_v, rows_v, out_v, sem):
    wid = lax.axis_index("s") * NC + lax.axis_index("c")
    tbase = wid * TPW
    pltpu.sync_copy(slots_h.at[pl.ds(tbase, TPW)], myslots_v)
    pltpu.sync_copy(gval_h.at[pl.ds(tbase, TPW)], gval_v)
    pltpu.sync_copy(lgval_h.at[pl.ds(tbase, TPW)], lgval_v)
    pltpu.async_copy(fs_h.at[myslots_v], rows_v, sem).wait()
    lane = lax.iota(jnp.int32, L)
    for k in range(TPW // L):
        tloc = k * L + lane
        g16 = gval_v[pl.ds(k * L, L)]
        lg16 = lgval_v[pl.ds(k * L, L)]
        for j in range(OUT):
            val = plsc.load_gather(rows_v, [tloc, lane * 0 + j])
            c = jnp.exp(val) * g16
            res = jnp.where(c == 0, _LOG_EPS, val + lg16)
            plsc.store_scatter(out_v, [tloc * OUT + j], res)
    pltpu.sync_copy(out_v, out_h.at[pl.ds(tbase * OUT, TPW * OUT)])


def _combine(fs, slots, gval, lgval):
    mesh = plsc.VectorSubcoreMesh(core_axis_name="c", subcore_axis_name="s")
    f = pl.kernel(
        _combine_body,
        compiler_params=pltpu.CompilerParams(needs_layout_passes=False),
        out_type=jax.ShapeDtypeStruct((T * OUT,), jnp.float32),
        mesh=mesh,
        scratch_types=[
            pltpu.VMEM((TPW,), jnp.int32),
            pltpu.VMEM((TPW,), jnp.float32),
            pltpu.VMEM((TPW,), jnp.float32),
            pltpu.VMEM((TPW, FW), jnp.float32),
            pltpu.VMEM((TPW * OUT,), jnp.float32),
            pltpu.SemaphoreType.DMA,
        ],
    )
    return f(fs, slots, gval, lgval)


_TRI = np.tril(np.ones((TB, TB), np.float32), -1)


def kernel(x, latent, gW1, gb1, gW2, gb2, gln_g, gln_b, gW3, gb3,
           eW0, eb0, eWh, ebh, eWs, ebs, eWo, ebo):
    tri = jnp.asarray(_TRI)
    eidx, gval, lgval, rank, astart, bexp = _gate(
        x, latent, gW1, gb1, gW2, gb2, gln_g, gln_b, gW3, gb3, tri)
    xls, slots = _dispatch(eidx, rank, astart, x, latent)
    fs = _grouped_mlp(bexp, xls,
                      eW0, eb0.reshape(E, 1, HID),
                      eWh, ebh, eWs,
                      ebs.reshape(E, 1, HID), eWo,
                      ebo.reshape(E, 1, OUT))
    out = _combine(fs, slots, gval, lgval)
    return out.reshape(T, OUT)


# speedup vs baseline: 3.5279x; 1.0225x over previous
import jax
import jax.numpy as jnp
import numpy as np
from jax import lax
from jax.experimental import pallas as pl
from jax.experimental.pallas import tpu as pltpu
from jax.experimental.pallas import tpu_sc as plsc

E = 8
K = 4
D_ENC = 256
D_LAT = 256
HID = 256
OUT = 4
DEPTH = 7
SKIP = 5
CH = D_LAT // E
XL = D_ENC + D_LAT

T = 4096
TB = 512
B = 256
LOGB = 8
NB = T // B + E
NBP = 32
P = NB * B
FW = 128

NC = 2
NS = 16
NW = NC * NS
TPW = T // NW
SR = 64
L = 16

_EPS = np.float32(np.finfo(np.float32).eps)
_LOG_EPS = np.float32(np.log(np.finfo(np.float32).eps))


def _dot(a, b):
    return jnp.dot(a, b, preferred_element_type=jnp.float32)


def _vgather(vec, idx):
    return lax.gather(
        vec, idx[:, None],
        lax.GatherDimensionNumbers(offset_dims=(),
                                   collapsed_slice_dims=(0,),
                                   start_index_map=(0,)),
        slice_sizes=(1,),
        mode=lax.GatherScatterMode.PROMISE_IN_BOUNDS)


def _gate_body(x_ref, lat_ref, gW1, gb1, gW2, gb2, gln_g, gln_b, gW3, gb3,
               tri_ref, pk_ref, gv2_ref, cnt_ref, bexp_ref, run_ref):
    i = pl.program_id(0)
    @pl.when(i == 0)
    def _():
        run_ref[...] = jnp.zeros((1, E), jnp.float32)
    xb = x_ref[...]
    lb = lat_ref[...]
    g = jax.nn.relu(_dot(xb, gW1[:D_ENC]) + _dot(lb, gW1[D_ENC:]) + gb1[...])
    g = _dot(g, gW2[...]) + gb2[...]
    m = g.mean(-1, keepdims=True)
    v = ((g - m) ** 2).mean(-1, keepdims=True)
    g = (g - m) / jnp.sqrt(v + 1e-5) * gln_g[...] + gln_b[...]
    logits = _dot(g, gW3[...]) + gb3[...]
    mx = logits.max(-1, keepdims=True)
    s = jnp.exp(logits - mx)
    s = s / s.sum(-1, keepdims=True)
    lanes = jax.lax.broadcasted_iota(jnp.int32, s.shape, 1)
    cur = s
    sum4 = jnp.zeros((s.shape[0], 1), jnp.float32)
    eidx = gmax = None
    for r in range(K):
        m_r = cur.max(-1, keepdims=True)
        sum4 = sum4 + m_r
        if r == 0:
            eidx = jnp.where(cur == m_r, lanes, E).min(-1, keepdims=True)
            gmax = m_r
        cur = jnp.where(cur == m_r, -jnp.inf, cur)
    gval = gmax / (sum4 + 1e-9)

    onehot = (lanes == eidx).astype(jnp.float32)
    prefix = _dot(tri_ref[...], onehot)
    run = run_ref[...]
    rank = ((prefix + run) * onehot).sum(-1, keepdims=True)
    newrun = run + onehot.sum(0, keepdims=True)
    run_ref[...] = newrun

    pk = (eidx << 12) | rank.astype(jnp.int32)
    pk_ref[...] = pk.reshape(1, TB, 1)
    gv2_ref[...] = jnp.concatenate([gval, jnp.log(gval)],
                                   axis=-1).reshape(1, TB, 2)

    @pl.when(i == pl.num_programs(0) - 1)
    def _():
        cnt = newrun.astype(jnp.int32)
        padded = ((cnt + (B - 1)) >> LOGB) << LOGB
        cum = padded
        for sh in (1, 2, 4):
            cum = cum + jnp.concatenate(
                [jnp.zeros((1, sh), jnp.int32), cum[:, :-sh]], axis=-1)
        astart = cum - padded
        cnt_ref[...] = jnp.concatenate(
            [astart, jnp.zeros((1, NBP - E), jnp.int32)], axis=-1)
        bstart = jax.lax.broadcasted_iota(jnp.int32, (1, NBP), 1) << LOGB
        acc = jnp.zeros((1, NBP), jnp.int32)
        for e in range(E):
            acc = acc + jnp.where(bstart >= cum[0, e], 1, 0)
        enc = jnp.where(bstart < cum[0, E - 1],
                        jnp.minimum(acc, E - 1) + E, E - 1)
        bexp_ref[...] = enc


def _gate(x, latent, gW1, gb1, gW2, gb2, gln_g, gln_b, gW3, gb3, tri):
    tok = lambda i: (i, 0)
    out3 = lambda i: (i, 0, 0)
    def wspec(a):
        return pl.BlockSpec(a.shape, lambda i, _a=a: tuple([0] * _a.ndim))
    res = pl.pallas_call(
        _gate_body,
        grid=(T // TB,),
        in_specs=[pl.BlockSpec((TB, D_ENC), tok),
                  pl.BlockSpec((TB, D_LAT), tok)]
                 + [wspec(a) for a in (gW1, gb1, gW2, gb2, gln_g, gln_b,
                                       gW3, gb3)]
                 + [pl.BlockSpec((TB, TB), lambda i: (0, 0))],
        out_specs=[pl.BlockSpec((1, TB, 1), out3),
                   pl.BlockSpec((1, TB, 2), out3),
                   pl.BlockSpec((1, NBP), lambda i: (0, 0)),
                   pl.BlockSpec((1, NBP), lambda i: (0, 0))],
        out_shape=[jax.ShapeDtypeStruct((T // TB, TB, 1), jnp.int32),
                   jax.ShapeDtypeStruct((T // TB, TB, 2), jnp.float32),
                   jax.ShapeDtypeStruct((1, NBP), jnp.int32),
                   jax.ShapeDtypeStruct((1, NBP), jnp.int32)],
        scratch_shapes=[pltpu.VMEM((1, E), jnp.float32)],
    )(x, latent, gW1, gb1, gW2, gb2, gln_g, gln_b, gW3, gb3, tri)
    pk, gv2, astart, bexp = res
    return (pk.reshape(T), gv2.reshape(T, 2),
            astart.reshape(NBP), bexp.reshape(NBP))


def _dispatch_body(pk_h, astart_h, x_h, lat_h,
                   xls_h, slots_h,
                   pk_v, astart_v, slots_lin_v, rows_v,
                   sem, sem2):
    wid = lax.axis_index("s") * NC + lax.axis_index("c")
    tbase = wid * TPW

    cpx = pltpu.async_copy(x_h.at[pl.ds(tbase, TPW)],
                           rows_v.at[:, 0, pl.ds(0, D_ENC)], sem)
    cpl = pltpu.async_copy(lat_h.at[pl.ds(tbase, TPW)],
                           rows_v.at[:, 0, pl.ds(D_ENC, D_LAT)], sem2)
    pltpu.sync_copy(pk_h.at[pl.ds(tbase, TPW)], pk_v)
    pltpu.sync_copy(astart_h.at[pl.ds(0, L)], astart_v)
    astart = astart_v[pl.ds(0, L)]
    for k in range(TPW // L):
        pk = pk_v[pl.ds(k * L, L)]
        base = _vgather(astart, pk >> 12)
        slot = base + (pk & 4095)
        slots_lin_v[pl.ds(k * L, L)] = slot
    pltpu.sync_copy(slots_lin_v, slots_h.at[pl.ds(tbase, TPW)])

    cpx.wait()
    cpl.wait()
    pltpu.async_copy(rows_v, xls_h.at[slots_lin_v], sem).wait()


def _dispatch(pk, astart, x, latent):
    mesh = plsc.VectorSubcoreMesh(core_axis_name="c", subcore_axis_name="s")
    f = pl.kernel(
        _dispatch_body,
        compiler_params=pltpu.CompilerParams(needs_layout_passes=False),
        out_type=[jax.ShapeDtypeStruct((P, 1, XL), jnp.float32),
                  jax.ShapeDtypeStruct((T,), jnp.int32)],
        mesh=mesh,
        scratch_types=[
            pltpu.VMEM((TPW,), jnp.int32),
            pltpu.VMEM((L,), jnp.int32),
            pltpu.VMEM((TPW,), jnp.int32),
            pltpu.VMEM((TPW, 1, XL), jnp.float32),
            pltpu.SemaphoreType.DMA,
            pltpu.SemaphoreType.DMA,
        ],
    )
    return f(pk, astart, x, latent)


def _mlp_body(bexp_sm, xls_ref, eW0, eb0, eWh, ebh, eWs, ebs,
              eWo, ebo, out_ref):
    enc = bexp_sm[pl.program_id(0)]
    @pl.when(enc >= E)
    def _():
        e = enc - E
        xls = xls_ref[:, 0, :]
        sel = (jax.lax.broadcasted_iota(jnp.int32, (D_LAT, CH), 0)
               == e * CH + jax.lax.broadcasted_iota(jnp.int32, (D_LAT, CH), 1)
               ).astype(jnp.float32)
        B2 = B // 2
        halves = [xls[:B2], xls[B2:]]
        chunks = [_dot(hh[:, D_ENC:], sel) for hh in halves]
        h0s = [jnp.concatenate([hh[:, :D_ENC], ck], axis=-1)
               for hh, ck in zip(halves, chunks)]
        hs = [jax.nn.relu(_dot(h0, eW0[0]) + eb0[0]) for h0 in h0s]
        hidx = 0
        for i in range(1, DEPTH):
            if i == SKIP:
                hs = [jnp.concatenate([h, h0], axis=-1)
                      for h, h0 in zip(hs, h0s)]
                hs = [jax.nn.relu(_dot(h, eWs[0]) + ebs[0]) for h in hs]
            else:
                hs = [jax.nn.relu(_dot(h, eWh[0, hidx]) + ebh[0, hidx])
                      for h in hs]
                hidx += 1
        os_ = [_dot(h, eWo[0]) + ebo[0] for h in hs]
        o = jnp.concatenate(os_, axis=0)
        out_ref[...] = jnp.concatenate(
            [o, jnp.zeros((B, FW - OUT), jnp.float32)], axis=-1)


def _grouped_mlp(bexp, xls, eW0, eb0, eWh, ebh, eWs, ebs, eWo, ebo):
    def ws(a):
        nd = a.ndim - 1
        return pl.BlockSpec((1,) + a.shape[1:],
                            lambda i, be, _n=nd: (be[i] & (E - 1),)
                            + (0,) * _n)
    grid_spec = pltpu.PrefetchScalarGridSpec(
        num_scalar_prefetch=1,
        grid=(NB,),
        in_specs=[
            pl.BlockSpec((B, 1, XL), lambda i, be: (i, 0, 0)),
            ws(eW0), ws(eb0), ws(eWh), ws(ebh),
            ws(eWs), ws(ebs), ws(eWo), ws(ebo),
        ],
        out_specs=pl.BlockSpec((B, FW), lambda i, be: (i, 0)),
    )
    return pl.pallas_call(
        _mlp_body,
        grid_spec=grid_spec,
        out_shape=jax.ShapeDtypeStruct((P, FW), jnp.float32),
    )(bexp, xls, eW0, eb0, eWh, ebh, eWs, ebs, eWo, ebo)


def _combine_body(fs_h, slots_h, gv2_h, out_h,
                  myslots_v, gv2_v, rows_v, out_v, sem):
    wid = lax.axis_index("s") * NC + lax.axis_index("c")
    tbase = wid * TPW
    pltpu.sync_copy(slots_h.at[pl.ds(tbase, TPW)], myslots_v)
    pltpu.sync_copy(gv2_h.at[pl.ds(tbase, TPW)], gv2_v)
    pltpu.async_copy(fs_h.at[myslots_v], rows_v, sem).wait()
    lane = lax.iota(jnp.int32, L)
    for k in range(TPW // L):
        tloc = k * L + lane
        g16 = plsc.load_gather(gv2_v, [tloc, lane * 0])
        lg16 = plsc.load_gather(gv2_v, [tloc, lane * 0 + 1])
        for j in range(OUT):
            val = plsc.load_gather(rows_v, [tloc, lane * 0 + j])
            c = jnp.exp(val) * g16
            res = jnp.where(c == 0, _LOG_EPS, val + lg16)
            plsc.store_scatter(out_v, [tloc * OUT + j], res)
    pltpu.sync_copy(out_v, out_h.at[pl.ds(tbase * OUT, TPW * OUT)])


def _combine(fs, slots, gv2):
    mesh = plsc.VectorSubcoreMesh(core_axis_name="c", subcore_axis_name="s")
    f = pl.kernel(
        _combine_body,
        compiler_params=pltpu.CompilerParams(needs_layout_passes=False),
        out_type=jax.ShapeDtypeStruct((T * OUT,), jnp.float32),
        mesh=mesh,
        scratch_types=[
            pltpu.VMEM((TPW,), jnp.int32),
            pltpu.VMEM((TPW, 2), jnp.float32),
            pltpu.VMEM((TPW, FW), jnp.float32),
            pltpu.VMEM((TPW * OUT,), jnp.float32),
            pltpu.SemaphoreType.DMA,
        ],
    )
    return f(fs, slots, gv2)


_TRI = np.tril(np.ones((TB, TB), np.float32), -1)


def kernel(x, latent, gW1, gb1, gW2, gb2, gln_g, gln_b, gW3, gb3,
           eW0, eb0, eWh, ebh, eWs, ebs, eWo, ebo):
    tri = jnp.asarray(_TRI)
    pk, gv2, astart, bexp = _gate(
        x, latent, gW1, gb1, gW2, gb2, gln_g, gln_b, gW3, gb3, tri)
    xls, slots = _dispatch(pk, astart, x, latent)
    fs = _grouped_mlp(bexp, xls,
                      eW0, eb0.reshape(E, 1, HID),
                      eWh, ebh, eWs,
                      ebs.reshape(E, 1, HID), eWo,
                      ebo.reshape(E, 1, OUT))
    out = _combine(fs, slots, gv2)
    return out.reshape(T, OUT)


# speedup vs baseline: 3.6357x; 1.0306x over previous
import jax
import jax.numpy as jnp
import numpy as np
from jax import lax
from jax.experimental import pallas as pl
from jax.experimental.pallas import tpu as pltpu
from jax.experimental.pallas import tpu_sc as plsc

E = 8
K = 4
D_ENC = 256
D_LAT = 256
HID = 256
OUT = 4
DEPTH = 7
SKIP = 5
CH = D_LAT // E
XL = D_ENC + D_LAT

T = 4096
TB = 512
B = 256
LOGB = 8
NB = T // B + E
NBP = 32
P = NB * B
FW = 128

NC = 2
NS = 16
NW = NC * NS
TPW = T // NW
SR = 64
L = 16

_EPS = np.float32(np.finfo(np.float32).eps)
_LOG_EPS = np.float32(np.log(np.finfo(np.float32).eps))


def _dot(a, b):
    return jnp.dot(a, b, preferred_element_type=jnp.float32)


def _vgather(vec, idx):
    return lax.gather(
        vec, idx[:, None],
        lax.GatherDimensionNumbers(offset_dims=(),
                                   collapsed_slice_dims=(0,),
                                   start_index_map=(0,)),
        slice_sizes=(1,),
        mode=lax.GatherScatterMode.PROMISE_IN_BOUNDS)


def _gate_body(x_ref, lat_ref, gW1, gb1, gW2, gb2, gln_g, gln_b, gW3, gb3,
               tri_ref, pk_ref, gv2_ref, cnt_ref, bexp_ref, run_ref):
    i = pl.program_id(0)
    @pl.when(i == 0)
    def _():
        run_ref[...] = jnp.zeros((1, E), jnp.float32)
    xb = x_ref[...]
    lb = lat_ref[...]
    g = jax.nn.relu(_dot(xb, gW1[:D_ENC]) + _dot(lb, gW1[D_ENC:]) + gb1[...])
    g = _dot(g, gW2[...]) + gb2[...]
    m = g.mean(-1, keepdims=True)
    v = (g * g).mean(-1, keepdims=True) - m * m
    g = (g - m) / jnp.sqrt(v + 1e-5) * gln_g[...] + gln_b[...]
    logits = _dot(g, gW3[...]) + gb3[...]
    mx = logits.max(-1, keepdims=True)
    s = jnp.exp(logits - mx)
    s = s / s.sum(-1, keepdims=True)
    lanes = jax.lax.broadcasted_iota(jnp.int32, s.shape, 1)
    lanesf = lanes.astype(jnp.float32)
    cur = s
    sum4 = jnp.zeros((s.shape[0], 1), jnp.float32)
    eidx = gmax = onehot = None
    for r in range(K):
        m_r = cur.max(-1, keepdims=True)
        sum4 = sum4 + m_r
        if r == 0:
            onehot = (cur == m_r).astype(jnp.float32)
            eidx = (onehot * lanesf).sum(-1, keepdims=True).astype(jnp.int32)
            gmax = m_r
        cur = jnp.where(cur == m_r, -jnp.inf, cur)
    gval = gmax / (sum4 + 1e-9)
    prefix = _dot(tri_ref[...], onehot)
    run = run_ref[...]
    rank = ((prefix + run) * onehot).sum(-1, keepdims=True)
    newrun = run + onehot.sum(0, keepdims=True)
    run_ref[...] = newrun

    pk = (eidx << 12) | rank.astype(jnp.int32)
    pk_ref[...] = pk.reshape(1, TB, 1)
    gv2_ref[...] = jnp.concatenate([gval, jnp.log(gval)],
                                   axis=-1).reshape(1, TB, 2)

    @pl.when(i == pl.num_programs(0) - 1)
    def _():
        cnt = newrun.astype(jnp.int32)
        padded = ((cnt + (B - 1)) >> LOGB) << LOGB
        cum = padded
        for sh in (1, 2, 4):
            cum = cum + jnp.concatenate(
                [jnp.zeros((1, sh), jnp.int32), cum[:, :-sh]], axis=-1)
        astart = cum - padded
        cnt_ref[...] = jnp.concatenate(
            [astart, jnp.zeros((1, NBP - E), jnp.int32)], axis=-1)
        bstart = jax.lax.broadcasted_iota(jnp.int32, (1, NBP), 1) << LOGB
        acc = jnp.zeros((1, NBP), jnp.int32)
        for e in range(E):
            acc = acc + jnp.where(bstart >= cum[0, e], 1, 0)
        enc = jnp.where(bstart < cum[0, E - 1],
                        jnp.minimum(acc, E - 1) + E, E - 1)
        bexp_ref[...] = enc


def _gate(x, latent, gW1, gb1, gW2, gb2, gln_g, gln_b, gW3, gb3, tri):
    tok = lambda i: (i, 0)
    out3 = lambda i: (i, 0, 0)
    def wspec(a):
        return pl.BlockSpec(a.shape, lambda i, _a=a: tuple([0] * _a.ndim))
    res = pl.pallas_call(
        _gate_body,
        grid=(T // TB,),
        in_specs=[pl.BlockSpec((TB, D_ENC), tok),
                  pl.BlockSpec((TB, D_LAT), tok)]
                 + [wspec(a) for a in (gW1, gb1, gW2, gb2, gln_g, gln_b,
                                       gW3, gb3)]
                 + [pl.BlockSpec((TB, TB), lambda i: (0, 0))],
        out_specs=[pl.BlockSpec((1, TB, 1), out3),
                   pl.BlockSpec((1, TB, 2), out3),
                   pl.BlockSpec((1, NBP), lambda i: (0, 0)),
                   pl.BlockSpec((1, NBP), lambda i: (0, 0))],
        out_shape=[jax.ShapeDtypeStruct((T // TB, TB, 1), jnp.int32),
                   jax.ShapeDtypeStruct((T // TB, TB, 2), jnp.float32),
                   jax.ShapeDtypeStruct((1, NBP), jnp.int32),
                   jax.ShapeDtypeStruct((1, NBP), jnp.int32)],
        scratch_shapes=[pltpu.VMEM((1, E), jnp.float32)],
    )(x, latent, gW1, gb1, gW2, gb2, gln_g, gln_b, gW3, gb3, tri)
    pk, gv2, astart, bexp = res
    return (pk.reshape(T), gv2.reshape(T, 2),
            astart.reshape(NBP), bexp.reshape(NBP))


def _dispatch_body(pk_h, astart_h, x_h, lat_h,
                   xls_h, slots_h,
                   pk_v, astart_v, slots_lin_v, rows_v,
                   sem, sem2):
    wid = lax.axis_index("s") * NC + lax.axis_index("c")
    tbase = wid * TPW

    cpx = pltpu.async_copy(x_h.at[pl.ds(tbase, TPW)],
                           rows_v.at[:, 0, pl.ds(0, D_ENC)], sem)
    cpl = pltpu.async_copy(lat_h.at[pl.ds(tbase, TPW)],
                           rows_v.at[:, 0, pl.ds(D_ENC, D_LAT)], sem2)
    pltpu.sync_copy(pk_h.at[pl.ds(tbase, TPW)], pk_v)
    pltpu.sync_copy(astart_h.at[pl.ds(0, L)], astart_v)
    astart = astart_v[pl.ds(0, L)]
    for k in range(TPW // L):
        pk = pk_v[pl.ds(k * L, L)]
        base = _vgather(astart, pk >> 12)
        slot = base + (pk & 4095)
        slots_lin_v[pl.ds(k * L, L)] = slot
    pltpu.sync_copy(slots_lin_v, slots_h.at[pl.ds(tbase, TPW)])

    cpx.wait()
    cpl.wait()
    pltpu.async_copy(rows_v, xls_h.at[slots_lin_v], sem).wait()


def _dispatch(pk, astart, x, latent):
    mesh = plsc.VectorSubcoreMesh(core_axis_name="c", subcore_axis_name="s")
    f = pl.kernel(
        _dispatch_body,
        compiler_params=pltpu.CompilerParams(needs_layout_passes=False),
        out_type=[jax.ShapeDtypeStruct((P, 1, XL), jnp.float32),
                  jax.ShapeDtypeStruct((T,), jnp.int32)],
        mesh=mesh,
        scratch_types=[
            pltpu.VMEM((TPW,), jnp.int32),
            pltpu.VMEM((L,), jnp.int32),
            pltpu.VMEM((TPW,), jnp.int32),
            pltpu.VMEM((TPW, 1, XL), jnp.float32),
            pltpu.SemaphoreType.DMA,
            pltpu.SemaphoreType.DMA,
        ],
    )
    return f(pk, astart, x, latent)


def _mlp_body(bexp_sm, xls_ref, eW0, eb0, eWh, ebh, eWs, ebs,
              eWo, ebo, out_ref):
    enc = bexp_sm[pl.program_id(0)]
    @pl.when(enc >= E)
    def _():
        e = enc - E
        xls = xls_ref[:, 0, :]
        sel = (jax.lax.broadcasted_iota(jnp.int32, (D_LAT, CH), 0)
               == e * CH + jax.lax.broadcasted_iota(jnp.int32, (D_LAT, CH), 1)
               ).astype(jnp.float32)
        B2 = B // 2
        halves = [xls[:B2], xls[B2:]]
        chunks = [_dot(hh[:, D_ENC:], sel) for hh in halves]
        h0s = [jnp.concatenate([hh[:, :D_ENC], ck], axis=-1)
               for hh, ck in zip(halves, chunks)]
        hs = [jax.nn.relu(_dot(h0, eW0[0]) + eb0[0]) for h0 in h0s]
        hidx = 0
        for i in range(1, DEPTH):
            if i == SKIP:
                hs = [jnp.concatenate([h, h0], axis=-1)
                      for h, h0 in zip(hs, h0s)]
                hs = [jax.nn.relu(_dot(h, eWs[0]) + ebs[0]) for h in hs]
            else:
                hs = [jax.nn.relu(_dot(h, eWh[0, hidx]) + ebh[0, hidx])
                      for h in hs]
                hidx += 1
        os_ = [_dot(h, eWo[0]) + ebo[0] for h in hs]
        o = jnp.concatenate(os_, axis=0)
        out_ref[...] = jnp.concatenate(
            [o, jnp.zeros((B, FW - OUT), jnp.float32)], axis=-1)


def _grouped_mlp(bexp, xls, eW0, eb0, eWh, ebh, eWs, ebs, eWo, ebo):
    def ws(a):
        nd = a.ndim - 1
        return pl.BlockSpec((1,) + a.shape[1:],
                            lambda i, be, _n=nd: (be[i] & (E - 1),)
                            + (0,) * _n)
    grid_spec = pltpu.PrefetchScalarGridSpec(
        num_scalar_prefetch=1,
        grid=(NB,),
        in_specs=[
            pl.BlockSpec((B, 1, XL), lambda i, be: (i, 0, 0)),
            ws(eW0), ws(eb0), ws(eWh), ws(ebh),
            ws(eWs), ws(ebs), ws(eWo), ws(ebo),
        ],
        out_specs=pl.BlockSpec((B, FW), lambda i, be: (i, 0)),
    )
    return pl.pallas_call(
        _mlp_body,
        grid_spec=grid_spec,
        out_shape=jax.ShapeDtypeStruct((P, FW), jnp.float32),
    )(bexp, xls, eW0, eb0, eWh, ebh, eWs, ebs, eWo, ebo)


def _combine_body(fs_h, slots_h, gv2_h, out_h,
                  myslots_v, gv2_v, rows_v, out_v, sem):
    wid = lax.axis_index("s") * NC + lax.axis_index("c")
    tbase = wid * TPW
    pltpu.sync_copy(slots_h.at[pl.ds(tbase, TPW)], myslots_v)
    pltpu.sync_copy(gv2_h.at[pl.ds(tbase, TPW)], gv2_v)
    pltpu.async_copy(fs_h.at[myslots_v], rows_v, sem).wait()
    lane = lax.iota(jnp.int32, L)
    for k in range(TPW // L):
        tloc = k * L + lane
        g16 = plsc.load_gather(gv2_v, [tloc, lane * 0])
        lg16 = plsc.load_gather(gv2_v, [tloc, lane * 0 + 1])
        for j in range(OUT):
            val = plsc.load_gather(rows_v, [tloc, lane * 0 + j])
            c = jnp.exp(val) * g16
            res = jnp.where(c == 0, _LOG_EPS, val + lg16)
            plsc.store_scatter(out_v, [tloc * OUT + j], res)
    pltpu.sync_copy(out_v, out_h.at[pl.ds(tbase * OUT, TPW * OUT)])


def _combine(fs, slots, gv2):
    mesh = plsc.VectorSubcoreMesh(core_axis_name="c", subcore_axis_name="s")
    f = pl.kernel(
        _combine_body,
        compiler_params=pltpu.CompilerParams(needs_layout_passes=False),
        out_type=jax.ShapeDtypeStruct((T * OUT,), jnp.float32),
        mesh=mesh,
        scratch_types=[
            pltpu.VMEM((TPW,), jnp.int32),
            pltpu.VMEM((TPW, 2), jnp.float32),
            pltpu.VMEM((TPW, FW), jnp.float32),
            pltpu.VMEM((TPW * OUT,), jnp.float32),
            pltpu.SemaphoreType.DMA,
        ],
    )
    return f(fs, slots, gv2)


_TRI = np.tril(np.ones((TB, TB), np.float32), -1)


def kernel(x, latent, gW1, gb1, gW2, gb2, gln_g, gln_b, gW3, gb3,
           eW0, eb0, eWh, ebh, eWs, ebs, eWo, ebo):
    tri = jnp.asarray(_TRI)
    pk, gv2, astart, bexp = _gate(
        x, latent, gW1, gb1, gW2, gb2, gln_g, gln_b, gW3, gb3, tri)
    xls, slots = _dispatch(pk, astart, x, latent)
    fs = _grouped_mlp(bexp, xls,
                      eW0, eb0.reshape(E, 1, HID),
                      eWh, ebh, eWs,
                      ebs.reshape(E, 1, HID), eWo,
                      ebo.reshape(E, 1, OUT))
    out = _combine(fs, slots, gv2)
    return out.reshape(T, OUT)
